# baseline (speedup 1.0000x reference)
import jax
import jax.numpy as jnp
from jax.experimental import pallas as pl
from jax.experimental.pallas import tpu as pltpu

T = 128
H = 768
FF = 2048
E = 16
K = 2

FFC = 2048
NC = FF // FFC
NSTEPS = E * NC
NBUF = 3


def _moe_kernel(x_ref, gate_w_ref, wg_hbm, wu_hbm, wd_hbm, out_ref,
                wg_buf, wu_buf, wd_buf, acc_ref, comb_ref, sems):
    x = x_ref[...]
    logits = jax.lax.dot_general(
        x, gate_w_ref[...], (((1,), (1,)), ((), ())),
        preferred_element_type=jnp.float32)
    idx = jax.lax.broadcasted_iota(jnp.int32, (T, E), 1)
    m1 = jnp.max(logits, axis=1, keepdims=True)
    i1 = jnp.min(jnp.where(logits == m1, idx, E), axis=1, keepdims=True)
    masked = jnp.where(idx == i1, -jnp.inf, logits)
    m2 = jnp.max(masked, axis=1, keepdims=True)
    i2 = jnp.min(jnp.where(masked == m2, idx, E), axis=1, keepdims=True)
    z = jnp.exp(m2 - m1)
    w1 = 1.0 / (1.0 + z)
    w2 = z / (1.0 + z)
    comb_ref[...] = jnp.where(idx == i1, w1, 0.0) + jnp.where(idx == i2, w2, 0.0)

    acc_ref[...] = jnp.zeros_like(acc_ref)

    def start_copies(step):
        slot = jax.lax.rem(step, NBUF)
        e = jax.lax.div(step, NC)
        c = jax.lax.rem(step, NC)
        pltpu.make_async_copy(
            wg_hbm.at[e, pl.ds(c * FFC, FFC), :], wg_buf.at[slot],
            sems.at[slot, 0]).start()
        pltpu.make_async_copy(
            wu_hbm.at[e, pl.ds(c * FFC, FFC), :], wu_buf.at[slot],
            sems.at[slot, 1]).start()
        pltpu.make_async_copy(
            wd_hbm.at[e, :, pl.ds(c * FFC, FFC)], wd_buf.at[slot],
            sems.at[slot, 2]).start()

    def wait_copies(step):
        slot = jax.lax.rem(step, NBUF)
        e = jax.lax.div(step, NC)
        c = jax.lax.rem(step, NC)
        pltpu.make_async_copy(
            wg_hbm.at[e, pl.ds(c * FFC, FFC), :], wg_buf.at[slot],
            sems.at[slot, 0]).wait()
        pltpu.make_async_copy(
            wu_hbm.at[e, pl.ds(c * FFC, FFC), :], wu_buf.at[slot],
            sems.at[slot, 1]).wait()
        pltpu.make_async_copy(
            wd_hbm.at[e, :, pl.ds(c * FFC, FFC)], wd_buf.at[slot],
            sems.at[slot, 2]).wait()

    for s in range(NBUF - 1):
        start_copies(s)

    def body(i, _):
        slot = jax.lax.rem(i, NBUF)
        e = jax.lax.div(i, NC)
        wait_copies(i)

        @pl.when(i + NBUF - 1 < NSTEPS)
        def _prefetch():
            start_copies(i + NBUF - 1)

        hg = jax.lax.dot_general(
            x, wg_buf[slot], (((1,), (1,)), ((), ())),
            preferred_element_type=jnp.float32)
        hu = jax.lax.dot_general(
            x, wu_buf[slot], (((1,), (1,)), ((), ())),
            preferred_element_type=jnp.float32)
        h = (hg * jax.lax.logistic(hg)) * hu
        y = jax.lax.dot_general(
            h, wd_buf[slot], (((1,), (1,)), ((), ())),
            preferred_element_type=jnp.float32)
        lane = jax.lax.broadcasted_iota(jnp.int32, (T, E), 1)
        cw = jnp.sum(jnp.where(lane == e, comb_ref[...], 0.0),
                     axis=1, keepdims=True)
        acc_ref[...] += cw * y
        return 0

    jax.lax.fori_loop(0, NSTEPS, body, 0)
    out_ref[...] = acc_ref[...]


@jax.jit
def kernel(x, gate_w, w_gate, w_up, w_down):
    return pl.pallas_call(
        _moe_kernel,
        in_specs=[
            pl.BlockSpec(memory_space=pltpu.VMEM),
            pl.BlockSpec(memory_space=pltpu.VMEM),
            pl.BlockSpec(memory_space=pl.ANY),
            pl.BlockSpec(memory_space=pl.ANY),
            pl.BlockSpec(memory_space=pl.ANY),
        ],
        out_specs=pl.BlockSpec(memory_space=pltpu.VMEM),
        out_shape=jax.ShapeDtypeStruct((T, H), jnp.float32),
        scratch_shapes=[
            pltpu.VMEM((NBUF, FFC, H), jnp.float32),
            pltpu.VMEM((NBUF, FFC, H), jnp.float32),
            pltpu.VMEM((NBUF, H, FFC), jnp.float32),
            pltpu.VMEM((T, H), jnp.float32),
            pltpu.VMEM((T, E), jnp.float32),
            pltpu.SemaphoreType.DMA((NBUF, 3)),
        ],
    )(x, gate_w, w_gate, w_up, w_down)


# speedup vs baseline: 1.0470x; 1.0470x over previous
import jax
import jax.numpy as jnp
from jax.experimental import pallas as pl
from jax.experimental.pallas import tpu as pltpu

T = 128
H = 768
FF = 2048
E = 16
K = 2

FFC = 1024
NC = FF // FFC
NSTEPS = E * NC
NBUF = 4
LOOKAHEAD = 2


def _moe_kernel(x_ref, gate_w_ref, wg_hbm, wu_hbm, wd_hbm, out_ref,
                wg_buf, wu_buf, wd_buf, h_ring, acc_ref, comb_ref, sems):
    x = x_ref[...]
    logits = jax.lax.dot_general(
        x, gate_w_ref[...], (((1,), (1,)), ((), ())),
        preferred_element_type=jnp.float32)
    idx = jax.lax.broadcasted_iota(jnp.int32, (T, E), 1)
    m1 = jnp.max(logits, axis=1, keepdims=True)
    i1 = jnp.min(jnp.where(logits == m1, idx, E), axis=1, keepdims=True)
    masked = jnp.where(idx == i1, -jnp.inf, logits)
    m2 = jnp.max(masked, axis=1, keepdims=True)
    i2 = jnp.min(jnp.where(masked == m2, idx, E), axis=1, keepdims=True)
    z = jnp.exp(m2 - m1)
    w1 = 1.0 / (1.0 + z)
    w2 = z / (1.0 + z)
    comb_ref[...] = jnp.where(idx == i1, w1, 0.0) + jnp.where(idx == i2, w2, 0.0)

    acc_ref[...] = jnp.zeros_like(acc_ref)

    def start_copies(step):
        slot = jax.lax.rem(step, NBUF)
        e = jax.lax.div(step, NC)
        c = jax.lax.rem(step, NC)
        pltpu.make_async_copy(
            wg_hbm.at[e, pl.ds(c * FFC, FFC), :], wg_buf.at[slot],
            sems.at[slot, 0]).start()
        pltpu.make_async_copy(
            wu_hbm.at[e, pl.ds(c * FFC, FFC), :], wu_buf.at[slot],
            sems.at[slot, 1]).start()
        pltpu.make_async_copy(
            wd_hbm.at[e, :, pl.ds(c * FFC, FFC)], wd_buf.at[slot],
            sems.at[slot, 2]).start()

    def wait_copies(step):
        slot = jax.lax.rem(step, NBUF)
        e = jax.lax.div(step, NC)
        c = jax.lax.rem(step, NC)
        pltpu.make_async_copy(
            wg_hbm.at[e, pl.ds(c * FFC, FFC), :], wg_buf.at[slot],
            sems.at[slot, 0]).wait()
        pltpu.make_async_copy(
            wu_hbm.at[e, pl.ds(c * FFC, FFC), :], wu_buf.at[slot],
            sems.at[slot, 1]).wait()
        pltpu.make_async_copy(
            wd_hbm.at[e, :, pl.ds(c * FFC, FFC)], wd_buf.at[slot],
            sems.at[slot, 2]).wait()

    for s in range(LOOKAHEAD):
        start_copies(s)

    lane = jax.lax.broadcasted_iota(jnp.int32, (T, E), 1)

    def body(i, _):
        @pl.when(i < NSTEPS)
        def _front():
            slot = jax.lax.rem(i, NBUF)
            e = jax.lax.div(i, NC)
            wait_copies(i)

            @pl.when(i + LOOKAHEAD < NSTEPS)
            def _prefetch():
                start_copies(i + LOOKAHEAD)

            hg = jax.lax.dot_general(
                x, wg_buf[slot], (((1,), (1,)), ((), ())),
                preferred_element_type=jnp.float32)
            hu = jax.lax.dot_general(
                x, wu_buf[slot], (((1,), (1,)), ((), ())),
                preferred_element_type=jnp.float32)
            cw = jnp.sum(jnp.where(lane == e, comb_ref[...], 0.0),
                         axis=1, keepdims=True)
            h_ring[jax.lax.rem(i, 2)] = (hg * jax.lax.logistic(hg)) * hu * cw

        @pl.when(i >= 1)
        def _back():
            j = i - 1
            slot_j = jax.lax.rem(j, NBUF)
            y = jax.lax.dot_general(
                h_ring[jax.lax.rem(j, 2)], wd_buf[slot_j],
                (((1,), (1,)), ((), ())),
                preferred_element_type=jnp.float32)
            acc_ref[...] += y

        return 0

    jax.lax.fori_loop(0, NSTEPS + 1, body, 0)
    out_ref[...] = acc_ref[...]


@jax.jit
def kernel(x, gate_w, w_gate, w_up, w_down):
    return pl.pallas_call(
        _moe_kernel,
        in_specs=[
            pl.BlockSpec(memory_space=pltpu.VMEM),
            pl.BlockSpec(memory_space=pltpu.VMEM),
            pl.BlockSpec(memory_space=pl.ANY),
            pl.BlockSpec(memory_space=pl.ANY),
            pl.BlockSpec(memory_space=pl.ANY),
        ],
        out_specs=pl.BlockSpec(memory_space=pltpu.VMEM),
        out_shape=jax.ShapeDtypeStruct((T, H), jnp.float32),
        scratch_shapes=[
            pltpu.VMEM((NBUF, FFC, H), jnp.float32),
            pltpu.VMEM((NBUF, FFC, H), jnp.float32),
            pltpu.VMEM((NBUF, H, FFC), jnp.float32),
            pltpu.VMEM((2, T, FFC), jnp.float32),
            pltpu.VMEM((T, H), jnp.float32),
            pltpu.VMEM((T, E), jnp.float32),
            pltpu.SemaphoreType.DMA((NBUF, 3)),
        ],
    )(x, gate_w, w_gate, w_up, w_down)
